# baseline (speedup 1.0000x reference)
import functools

import jax
import jax.numpy as jnp
from jax import lax
from jax.experimental import pallas as pl
from jax.experimental.pallas import tpu as pltpu
from jax.experimental.pallas import tpu_sc as plsc

BS = 4
SEQ = 4096
EMB = 2048

S_SPLIT = 3072
_UNROLL = 4

_KS0 = 0
_KS1 = 42
_KS2 = (0x1BD11BDA ^ 42)
_ROT_A = (13, 15, 26, 6)
_ROT_B = (17, 29, 16, 24)


def _rotl(x, r):
    return lax.shift_left(x, r) | lax.shift_right_logical(x, 32 - r)


def _threefry_tail(x0, x1):
    def rounds(x0, x1, rots):
        for r in rots:
            x0 = x0 + x1
            x1 = x0 ^ _rotl(x1, r)
        return x0, x1

    x0, x1 = rounds(x0, x1, _ROT_A[1:])
    x0 = x0 + _KS1
    x1 = x1 + (_KS2 + 1)
    x0, x1 = rounds(x0, x1, _ROT_B)
    x0 = x0 + _KS2
    x1 = x1 + (_KS0 + 2)
    x0, x1 = rounds(x0, x1, _ROT_A)
    x0 = x0 + _KS0
    x1 = x1 + (_KS1 + 3)
    x0, x1 = rounds(x0, x1, _ROT_B)
    x0 = x0 + _KS1
    x1 = x1 + (_KS2 + 4)
    x0, x1 = rounds(x0, x1, _ROT_A)
    x0 = x0 + _KS2
    x1 = x1 + (_KS0 + 5)
    return x0 ^ x1


def _threefry_bits(i_vec):
    x1i = i_vec + _KS1
    x0 = x1i
    x1 = x0 ^ _rotl(x1i, 13)
    return _threefry_tail(x0, x1)


_U = 8


def _tc_sample_body(s_ref, m_ref):
    b = pl.program_id(0)
    jrow = lax.broadcasted_iota(jnp.int32, (16, 128), 0)
    jlan = lax.broadcasted_iota(jnp.int32, (16, 128), 1)
    base = (b * EMB + jrow * 128 + jlan) * SEQ

    def group(best, sbest, s0):
        V = base + (s0 + _KS1)
        R = _rotl(V, 13)
        W = V ^ R
        posts = []
        for u in range(_U):
            if u <= 5:
                d = 2 ^ (2 + u)
                D = d | (d << 13)
                x0 = (V ^ d) if d else V
                x1 = (W ^ D) if D else W
            else:
                x1i = V + u
                x0 = x1i
                x1 = x0 ^ _rotl(x1i, 13)
            posts.append((x0, x1))
        mants = [
            lax.shift_right_logical(_threefry_tail(x0, x1), 9)
            for (x0, x1) in posts
        ]
        ms = list(mants)
        ss = [s0 + u for u in range(_U)]
        n = _U
        while n > 1:
            nms, nss = [], []
            for k in range(0, n, 2):
                nms.append(jnp.maximum(ms[k], ms[k + 1]))
                nss.append(jnp.where(ms[k + 1] > ms[k], ss[k + 1], ss[k]))
            ms, ss = nms, nss
            n //= 2
        m, s = ms[0], ss[0]
        cond = m > best
        best = jnp.where(cond, m, best)
        sbest = jnp.where(cond, s, sbest)
        return best, sbest

    def step(t, carry):
        best, sbest = carry
        s0 = t * (_UNROLL * _U)
        for q in range(_UNROLL):
            best, sbest = group(best, sbest, s0 + q * _U)
        return best, sbest

    best = jnp.full((16, 128), -1, jnp.int32)
    sbest = jnp.zeros((16, 128), jnp.int32)
    best, sbest = lax.fori_loop(0, S_SPLIT // (_UNROLL * _U), step,
                                (best, sbest))
    s_ref[...] = sbest
    m_ref[...] = best


def _tc_sample():
    return pl.pallas_call(
        _tc_sample_body,
        grid=(BS,),
        out_specs=[
            pl.BlockSpec((16, 128), lambda b: (b, 0)),
            pl.BlockSpec((16, 128), lambda b: (b, 0)),
        ],
        out_shape=[
            jax.ShapeDtypeStruct((BS * 16, 128), jnp.int32),
            jax.ShapeDtypeStruct((BS * 16, 128), jnp.int32),
        ],
    )()


_NW = 32
_PW = (BS * EMB) // _NW
_CH = 128
_G = 8


def _sc_sample_body(s_out, m_out, bufs, bufm, sem):
    wid = lax.axis_index("s") * 2 + lax.axis_index("c")
    pbase = wid * _PW
    b = lax.shift_right_logical(wid, 3)
    jpos = (wid & 7) * _PW
    lane = lax.iota(jnp.int32, 16)
    for og in range(_PW // (16 * _G)):
        bases = [
            (b * EMB + jpos + (og * _G + g) * 16 + lane) * SEQ
            for g in range(_G)
        ]

        def step(t, carry):
            news = []
            for g in range(_G):
                best, sbest = carry[2 * g], carry[2 * g + 1]
                mant = lax.shift_right_logical(
                    _threefry_bits(bases[g] + t), 9)
                cond = mant > best
                news.append(jnp.where(cond, mant, best))
                news.append(jnp.where(cond, t, sbest))
            return tuple(news)

        init = []
        for g in range(_G):
            init.append(jnp.full((16,), -1, jnp.int32))
            init.append(jnp.zeros((16,), jnp.int32))
        res = lax.fori_loop(S_SPLIT, SEQ, step, tuple(init))
        for g in range(_G):
            bufm[pl.ds((og * _G + g) * 16, 16)] = res[2 * g]
            bufs[pl.ds((og * _G + g) * 16, 16)] = res[2 * g + 1]
    pltpu.sync_copy(bufs, s_out.at[pl.ds(pbase, _PW)])
    pltpu.sync_copy(bufm, m_out.at[pl.ds(pbase, _PW)])


def _sc_sample():
    mesh = plsc.VectorSubcoreMesh(core_axis_name="c", subcore_axis_name="s")
    f = functools.partial(
        pl.kernel,
        mesh=mesh,
        out_type=[
            jax.ShapeDtypeStruct((BS * EMB,), jnp.int32),
            jax.ShapeDtypeStruct((BS * EMB,), jnp.int32),
        ],
        scratch_types=[
            pltpu.VMEM((_PW,), jnp.int32),
            pltpu.VMEM((_PW,), jnp.int32),
            pltpu.SemaphoreType.DMA,
        ],
    )(_sc_sample_body)
    return f()


def _gather_body(x_hbm, stc_hbm, mtc_hbm, ssc_hbm, msc_hbm, out_hbm,
                 stv, mtv, ssv, msv, a0, a1, buf0, buf1, rowsv, sem, sem2):
    wid = lax.axis_index("s") * 2 + lax.axis_index("c")
    pbase = wid * _PW
    b = lax.shift_right_logical(wid, 3)
    jpos = (wid & 7) * _PW
    cps = [
        pltpu.make_async_copy(stc_hbm.at[pl.ds(pbase, _PW)], stv, sem),
        pltpu.make_async_copy(mtc_hbm.at[pl.ds(pbase, _PW)], mtv, sem),
        pltpu.make_async_copy(ssc_hbm.at[pl.ds(pbase, _PW)], ssv, sem),
        pltpu.make_async_copy(msc_hbm.at[pl.ds(pbase, _PW)], msv, sem),
    ]
    for cp in cps:
        cp.start()
    for cp in cps:
        cp.wait()
    lane = lax.iota(jnp.int32, 16)
    rowoff = b * SEQ
    for g in range(_PW // 16):
        sl = pl.ds(g * 16, 16)
        v = jnp.where(mtv[sl] >= msv[sl], stv[sl], ssv[sl]) + rowoff
        if g < 8:
            a0[pl.ds(g * 16, 16)] = v
        else:
            a1[pl.ds((g - 8) * 16, 16)] = v
    col0 = pl.multiple_of(jpos, 128)
    col1 = pl.multiple_of(jpos + _CH, 128)
    cp0 = pltpu.make_async_copy(x_hbm.at[a0, pl.ds(col0, _CH)], buf0, sem)
    cp1 = pltpu.make_async_copy(x_hbm.at[a1, pl.ds(col1, _CH)], buf1, sem2)
    cp0.start()
    cp1.start()
    cp0.wait()
    for g in range(8):
        acc0 = jnp.zeros((16,), jnp.float32)
        for k in range(16):
            w = g * 16 + k
            acc0 = jnp.where(lane == k, buf0[w, pl.ds(g * 16, 16)], acc0)
        rowsv[pl.ds(g * 16, 16)] = acc0
    pltpu.sync_copy(rowsv.at[pl.ds(0, _CH)], out_hbm.at[b, pl.ds(col0, _CH)])
    cp1.wait()
    for g in range(8):
        acc1 = jnp.zeros((16,), jnp.float32)
        for k in range(16):
            w = g * 16 + k
            acc1 = jnp.where(lane == k, buf1[w, pl.ds(g * 16, 16)], acc1)
        rowsv[pl.ds(_CH + g * 16, 16)] = acc1
    pltpu.sync_copy(rowsv.at[pl.ds(_CH, _CH)], out_hbm.at[b, pl.ds(col1, _CH)])


def _gather(x2, s_tc, m_tc, s_sc, m_sc):
    mesh = plsc.VectorSubcoreMesh(core_axis_name="c", subcore_axis_name="s")
    f = functools.partial(
        pl.kernel,
        mesh=mesh,
        out_type=jax.ShapeDtypeStruct((BS, EMB), jnp.float32),
        scratch_types=[
            pltpu.VMEM((_PW,), jnp.int32),
            pltpu.VMEM((_PW,), jnp.int32),
            pltpu.VMEM((_PW,), jnp.int32),
            pltpu.VMEM((_PW,), jnp.int32),
            pltpu.VMEM((_CH,), jnp.int32),
            pltpu.VMEM((_CH,), jnp.int32),
            pltpu.VMEM((_CH, _CH), jnp.float32),
            pltpu.VMEM((_CH, _CH), jnp.float32),
            pltpu.VMEM((_PW,), jnp.float32),
            pltpu.SemaphoreType.DMA,
            pltpu.SemaphoreType.DMA,
        ],
    )(_gather_body)
    return f(x2, s_tc, m_tc, s_sc, m_sc)


def kernel(output_tokens_from_bert):
    x = output_tokens_from_bert
    s_sc, m_sc = _sc_sample()
    s_tc2, m_tc2 = _tc_sample()
    s_tc = s_tc2.reshape(BS * EMB)
    m_tc = m_tc2.reshape(BS * EMB)
    x2 = x.reshape(BS * SEQ, EMB)
    return _gather(x2, s_tc, m_tc, s_sc, m_sc)


# speedup vs baseline: 1.0227x; 1.0227x over previous
import functools

import jax
import jax.numpy as jnp
from jax import lax
from jax.experimental import pallas as pl
from jax.experimental.pallas import tpu as pltpu
from jax.experimental.pallas import tpu_sc as plsc

BS = 4
SEQ = 4096
EMB = 2048

S_SPLIT = 3104
_UNROLL = 4

_KS0 = 0
_KS1 = 42
_KS2 = (0x1BD11BDA ^ 42)
_ROT_A = (13, 15, 26, 6)
_ROT_B = (17, 29, 16, 24)


def _rotl(x, r):
    return lax.shift_left(x, r) | lax.shift_right_logical(x, 32 - r)


def _threefry_tail(x0, x1):
    def rounds(x0, x1, rots):
        for r in rots:
            x0 = x0 + x1
            x1 = x0 ^ _rotl(x1, r)
        return x0, x1

    x0, x1 = rounds(x0, x1, _ROT_A[1:])
    x0 = x0 + _KS1
    x1 = x1 + (_KS2 + 1)
    x0, x1 = rounds(x0, x1, _ROT_B)
    x0 = x0 + _KS2
    x1 = x1 + (_KS0 + 2)
    x0, x1 = rounds(x0, x1, _ROT_A)
    x0 = x0 + _KS0
    x1 = x1 + (_KS1 + 3)
    x0, x1 = rounds(x0, x1, _ROT_B)
    x0 = x0 + _KS1
    x1 = x1 + (_KS2 + 4)
    x0, x1 = rounds(x0, x1, _ROT_A)
    x0 = x0 + _KS2
    x1 = x1 + (_KS0 + 5)
    return x0 ^ x1


def _threefry_bits(i_vec):
    x1i = i_vec + _KS1
    x0 = x1i
    x1 = x0 ^ _rotl(x1i, 13)
    return _threefry_tail(x0, x1)


_U = 8


def _tc_sample_body(s_ref, m_ref):
    b = pl.program_id(0)
    jrow = lax.broadcasted_iota(jnp.int32, (16, 128), 0)
    jlan = lax.broadcasted_iota(jnp.int32, (16, 128), 1)
    base = (b * EMB + jrow * 128 + jlan) * SEQ

    def group(best, sbest, s0):
        V = base + (s0 + _KS1)
        R = _rotl(V, 13)
        W = V ^ R
        posts = []
        for u in range(_U):
            if u <= 5:
                d = 2 ^ (2 + u)
                D = d | (d << 13)
                x0 = (V ^ d) if d else V
                x1 = (W ^ D) if D else W
            else:
                x1i = V + u
                x0 = x1i
                x1 = x0 ^ _rotl(x1i, 13)
            posts.append((x0, x1))
        mants = [
            lax.shift_right_logical(_threefry_tail(x0, x1), 9)
            for (x0, x1) in posts
        ]
        ms = list(mants)
        ss = [s0 + u for u in range(_U)]
        n = _U
        while n > 1:
            nms, nss = [], []
            for k in range(0, n, 2):
                nms.append(jnp.maximum(ms[k], ms[k + 1]))
                nss.append(jnp.where(ms[k + 1] > ms[k], ss[k + 1], ss[k]))
            ms, ss = nms, nss
            n //= 2
        m, s = ms[0], ss[0]
        cond = m > best
        best = jnp.where(cond, m, best)
        sbest = jnp.where(cond, s, sbest)
        return best, sbest

    def step(t, carry):
        best, sbest = carry
        s0 = t * (_UNROLL * _U)
        for q in range(_UNROLL):
            best, sbest = group(best, sbest, s0 + q * _U)
        return best, sbest

    best = jnp.full((16, 128), -1, jnp.int32)
    sbest = jnp.zeros((16, 128), jnp.int32)
    best, sbest = lax.fori_loop(0, S_SPLIT // (_UNROLL * _U), step,
                                (best, sbest))
    s_ref[...] = sbest
    m_ref[...] = best


def _tc_sample():
    return pl.pallas_call(
        _tc_sample_body,
        grid=(BS,),
        out_specs=[
            pl.BlockSpec((16, 128), lambda b: (b, 0)),
            pl.BlockSpec((16, 128), lambda b: (b, 0)),
        ],
        out_shape=[
            jax.ShapeDtypeStruct((BS * 16, 128), jnp.int32),
            jax.ShapeDtypeStruct((BS * 16, 128), jnp.int32),
        ],
    )()


_NW = 32
_PW = (BS * EMB) // _NW
_CH = 128
_G = 8


def _sc_sample_body(s_out, m_out, bufs, bufm, sem):
    wid = lax.axis_index("s") * 2 + lax.axis_index("c")
    pbase = wid * _PW
    b = lax.shift_right_logical(wid, 3)
    jpos = (wid & 7) * _PW
    lane = lax.iota(jnp.int32, 16)
    for og in range(_PW // (16 * _G)):
        bases = [
            (b * EMB + jpos + (og * _G + g) * 16 + lane) * SEQ
            for g in range(_G)
        ]

        def step(t, carry):
            news = []
            for g in range(_G):
                best, sbest = carry[2 * g], carry[2 * g + 1]
                mant = lax.shift_right_logical(
                    _threefry_bits(bases[g] + t), 9)
                cond = mant > best
                news.append(jnp.where(cond, mant, best))
                news.append(jnp.where(cond, t, sbest))
            return tuple(news)

        init = []
        for g in range(_G):
            init.append(jnp.full((16,), -1, jnp.int32))
            init.append(jnp.zeros((16,), jnp.int32))
        res = lax.fori_loop(S_SPLIT, SEQ, step, tuple(init))
        for g in range(_G):
            bufm[pl.ds((og * _G + g) * 16, 16)] = res[2 * g]
            bufs[pl.ds((og * _G + g) * 16, 16)] = res[2 * g + 1]
    pltpu.sync_copy(bufs, s_out.at[pl.ds(pbase, _PW)])
    pltpu.sync_copy(bufm, m_out.at[pl.ds(pbase, _PW)])


def _sc_sample():
    mesh = plsc.VectorSubcoreMesh(core_axis_name="c", subcore_axis_name="s")
    f = functools.partial(
        pl.kernel,
        mesh=mesh,
        out_type=[
            jax.ShapeDtypeStruct((BS * EMB,), jnp.int32),
            jax.ShapeDtypeStruct((BS * EMB,), jnp.int32),
        ],
        scratch_types=[
            pltpu.VMEM((_PW,), jnp.int32),
            pltpu.VMEM((_PW,), jnp.int32),
            pltpu.SemaphoreType.DMA,
        ],
    )(_sc_sample_body)
    return f()


def _gather_body(x_hbm, stc_hbm, mtc_hbm, ssc_hbm, msc_hbm, out_hbm,
                 stv, mtv, ssv, msv, a0, a1, buf0, buf1, rowsv, sem, sem2):
    wid = lax.axis_index("s") * 2 + lax.axis_index("c")
    pbase = wid * _PW
    b = lax.shift_right_logical(wid, 3)
    jpos = (wid & 7) * _PW
    cps = [
        pltpu.make_async_copy(stc_hbm.at[pl.ds(pbase, _PW)], stv, sem),
        pltpu.make_async_copy(mtc_hbm.at[pl.ds(pbase, _PW)], mtv, sem),
        pltpu.make_async_copy(ssc_hbm.at[pl.ds(pbase, _PW)], ssv, sem),
        pltpu.make_async_copy(msc_hbm.at[pl.ds(pbase, _PW)], msv, sem),
    ]
    for cp in cps:
        cp.start()
    for cp in cps:
        cp.wait()
    lane = lax.iota(jnp.int32, 16)
    rowoff = b * SEQ
    for g in range(_PW // 16):
        sl = pl.ds(g * 16, 16)
        v = jnp.where(mtv[sl] >= msv[sl], stv[sl], ssv[sl]) + rowoff
        if g < 8:
            a0[pl.ds(g * 16, 16)] = v
        else:
            a1[pl.ds((g - 8) * 16, 16)] = v
    col0 = pl.multiple_of(jpos, 128)
    col1 = pl.multiple_of(jpos + _CH, 128)
    cp0 = pltpu.make_async_copy(x_hbm.at[a0, pl.ds(col0, _CH)], buf0, sem)
    cp1 = pltpu.make_async_copy(x_hbm.at[a1, pl.ds(col1, _CH)], buf1, sem2)
    cp0.start()
    cp1.start()
    cp0.wait()
    for g in range(8):
        acc0 = jnp.zeros((16,), jnp.float32)
        for k in range(16):
            w = g * 16 + k
            acc0 = jnp.where(lane == k, buf0[w, pl.ds(g * 16, 16)], acc0)
        rowsv[pl.ds(g * 16, 16)] = acc0
    pltpu.sync_copy(rowsv.at[pl.ds(0, _CH)], out_hbm.at[b, pl.ds(col0, _CH)])
    cp1.wait()
    for g in range(8):
        acc1 = jnp.zeros((16,), jnp.float32)
        for k in range(16):
            w = g * 16 + k
            acc1 = jnp.where(lane == k, buf1[w, pl.ds(g * 16, 16)], acc1)
        rowsv[pl.ds(_CH + g * 16, 16)] = acc1
    pltpu.sync_copy(rowsv.at[pl.ds(_CH, _CH)], out_hbm.at[b, pl.ds(col1, _CH)])


def _gather(x2, s_tc, m_tc, s_sc, m_sc):
    mesh = plsc.VectorSubcoreMesh(core_axis_name="c", subcore_axis_name="s")
    f = functools.partial(
        pl.kernel,
        mesh=mesh,
        out_type=jax.ShapeDtypeStruct((BS, EMB), jnp.float32),
        scratch_types=[
            pltpu.VMEM((_PW,), jnp.int32),
            pltpu.VMEM((_PW,), jnp.int32),
            pltpu.VMEM((_PW,), jnp.int32),
            pltpu.VMEM((_PW,), jnp.int32),
            pltpu.VMEM((_CH,), jnp.int32),
            pltpu.VMEM((_CH,), jnp.int32),
            pltpu.VMEM((_CH, _CH), jnp.float32),
            pltpu.VMEM((_CH, _CH), jnp.float32),
            pltpu.VMEM((_PW,), jnp.float32),
            pltpu.SemaphoreType.DMA,
            pltpu.SemaphoreType.DMA,
        ],
    )(_gather_body)
    return f(x2, s_tc, m_tc, s_sc, m_sc)


def kernel(output_tokens_from_bert):
    x = output_tokens_from_bert
    s_sc, m_sc = _sc_sample()
    s_tc2, m_tc2 = _tc_sample()
    s_tc = s_tc2.reshape(BS * EMB)
    m_tc = m_tc2.reshape(BS * EMB)
    x2 = x.reshape(BS * SEQ, EMB)
    return _gather(x2, s_tc, m_tc, s_sc, m_sc)
